# Initial kernel scaffold; baseline (speedup 1.0000x reference)
import functools

import jax
import jax.numpy as jnp
from jax import lax
from jax.experimental import pallas as pl
from jax.experimental.pallas import tpu as pltpu
from jax.experimental.pallas import tpu_sc as plsc

_ROW_BLK = 256


def _argmin_body(ms_ref, xt_ref, emb_ref, embt_ref, idx_ref, tbl_ref,
                 cn_ref):
    i = pl.program_id(0)
    mean = ms_ref[0, 0]
    std = ms_ref[0, 1]
    safe = jnp.where(std == 0.0, 1.0, std)

    @pl.when(i == 0)
    def _():
        emb = emb_ref[...]
        cn_ref[...] = jnp.sum(emb * emb, axis=0, keepdims=True)

    flat = jnp.where(std == 0.0, 0.0, xt_ref[...] / safe) - mean
    rn = jnp.sum(flat * flat, axis=1, keepdims=True)
    mm = jnp.dot(flat, emb_ref[...], preferred_element_type=jnp.float32)
    dist = (rn + cn_ref[...]) - 2.0 * mm
    m = jnp.min(dist, axis=1, keepdims=True)
    iota = lax.broadcasted_iota(jnp.int32, dist.shape, 1)
    big = jnp.int32(dist.shape[1])
    idx_ref[0, 0, :] = jnp.min(jnp.where(dist == m, iota, big), axis=1)

    tbl_ref[...] = (embt_ref[...] + mean) * std


def _tc_argmin(ms, xt, emb, embt):
    n_rows, emb_dim = xt.shape
    num_emb = emb.shape[1]
    grid = n_rows // _ROW_BLK
    return pl.pallas_call(
        _argmin_body,
        grid=(grid,),
        in_specs=[
            pl.BlockSpec(memory_space=pltpu.SMEM),
            pl.BlockSpec((_ROW_BLK, emb_dim), lambda i: (i, 0)),
            pl.BlockSpec((emb_dim, num_emb), lambda i: (0, 0)),
            pl.BlockSpec((_ROW_BLK, emb_dim), lambda i: (i, 0)),
        ],
        out_specs=[
            pl.BlockSpec((1, 1, _ROW_BLK), lambda i: (i, 0, 0)),
            pl.BlockSpec((_ROW_BLK, emb_dim), lambda i: (i, 0)),
        ],
        out_shape=[
            jax.ShapeDtypeStruct((grid, 1, _ROW_BLK), jnp.int32),
            jax.ShapeDtypeStruct((n_rows, emb_dim), jnp.float32),
        ],
        scratch_shapes=[pltpu.VMEM((1, num_emb), jnp.float32)],
        compiler_params=pltpu.CompilerParams(
            dimension_semantics=("arbitrary",),
        ),
    )(ms, xt, emb, embt)


def _sc_gather(table, idx):
    n_rows, emb_dim = table.shape
    mesh = plsc.VectorSubcoreMesh(core_axis_name="c", subcore_axis_name="s")
    n_workers = 32
    b_per_w = idx.shape[0] // n_workers

    @functools.partial(
        pl.kernel,
        mesh=mesh,
        out_type=jax.ShapeDtypeStruct((idx.shape[0], emb_dim), jnp.float32),
        scratch_types=[
            pltpu.VMEM((b_per_w,), jnp.int32),
            pltpu.VMEM((b_per_w, emb_dim), jnp.float32),
            pltpu.SemaphoreType.DMA,
        ],
    )
    def k(table_hbm, idx_hbm, out_hbm, idx_v, rows_v, sem):
        wid = lax.axis_index("s") * 2 + lax.axis_index("c")
        base = wid * b_per_w
        pltpu.sync_copy(idx_hbm.at[pl.ds(base, b_per_w)], idx_v)
        pltpu.async_copy(table_hbm.at[idx_v], rows_v, sem).wait()
        pltpu.sync_copy(rows_v, out_hbm.at[pl.ds(base, b_per_w)])

    return k(table, idx)


def kernel(x, mean, stddev, embeddings):
    b, h, w, c = x.shape
    emb_dim, num_emb = embeddings.shape
    xt = jnp.transpose(x, (0, 3, 1, 2)).reshape(-1, emb_dim)
    embt = embeddings.T
    ms = jnp.stack([mean[0], stddev[0]]).reshape(1, 2)
    idx3, table = _tc_argmin(ms, xt, embeddings, embt)
    idx = idx3.reshape(-1)
    q = _sc_gather(table, idx)
    quantised = q.reshape(b, c, h, w).transpose(0, 2, 3, 1)
    disc_out = idx.reshape(-1, c)
    return quantised, disc_out


# speedup vs baseline: 1.4170x; 1.4170x over previous
import functools

import jax
import jax.numpy as jnp
from jax import lax
from jax.experimental import pallas as pl
from jax.experimental.pallas import tpu as pltpu
from jax.experimental.pallas import tpu_sc as plsc

_ROW_BLK = 256


def _argmin_body(ms_ref, xt_ref, emb_ref, embt_ref, idx_ref, tbl_ref,
                 cn_ref):
    i = pl.program_id(0)
    mean = ms_ref[0, 0]
    std = ms_ref[0, 1]
    safe = jnp.where(std == 0.0, 1.0, std)

    @pl.when(i == 0)
    def _():
        emb = emb_ref[...]
        cn_ref[...] = jnp.sum(emb * emb, axis=0, keepdims=True)

    flat = jnp.where(std == 0.0, 0.0, xt_ref[...] / safe) - mean
    rn = jnp.sum(flat * flat, axis=1, keepdims=True)
    mm = jnp.dot(flat, emb_ref[...], preferred_element_type=jnp.float32)
    dist = (rn + cn_ref[...]) - 2.0 * mm
    m = jnp.min(dist, axis=1, keepdims=True)
    iota = lax.broadcasted_iota(jnp.int32, dist.shape, 1)
    big = jnp.int32(dist.shape[1])
    idx_ref[0, 0, :] = jnp.min(jnp.where(dist == m, iota, big), axis=1)

    tbl_ref[:, 0:32] = (embt_ref[...] + mean) * std


def _tc_argmin(ms, xt, emb, embt):
    n_rows, emb_dim = xt.shape
    num_emb = emb.shape[1]
    grid = n_rows // _ROW_BLK
    return pl.pallas_call(
        _argmin_body,
        grid=(grid,),
        in_specs=[
            pl.BlockSpec(memory_space=pltpu.SMEM),
            pl.BlockSpec((_ROW_BLK, emb_dim), lambda i: (i, 0)),
            pl.BlockSpec((emb_dim, num_emb), lambda i: (0, 0)),
            pl.BlockSpec((_ROW_BLK, emb_dim), lambda i: (i, 0)),
        ],
        out_specs=[
            pl.BlockSpec((1, 1, _ROW_BLK), lambda i: (i, 0, 0)),
            pl.BlockSpec((_ROW_BLK, 128), lambda i: (i, 0)),
        ],
        out_shape=[
            jax.ShapeDtypeStruct((grid, 1, _ROW_BLK), jnp.int32),
            jax.ShapeDtypeStruct((n_rows, 128), jnp.float32),
        ],
        scratch_shapes=[pltpu.VMEM((1, num_emb), jnp.float32)],
        compiler_params=pltpu.CompilerParams(
            dimension_semantics=("arbitrary",),
        ),
    )(ms, xt, emb, embt)


def _sc_gather(table, idx):
    n_rows, row_w = table.shape
    mesh = plsc.VectorSubcoreMesh(core_axis_name="c", subcore_axis_name="s")
    n_workers = 32
    b_per_w = idx.shape[0] // n_workers

    @functools.partial(
        pl.kernel,
        mesh=mesh,
        out_type=jax.ShapeDtypeStruct((idx.shape[0], row_w), jnp.float32),
        scratch_types=[
            pltpu.VMEM((b_per_w,), jnp.int32),
            pltpu.VMEM((b_per_w, row_w), jnp.float32),
            pltpu.SemaphoreType.DMA,
        ],
    )
    def k(table_hbm, idx_hbm, out_hbm, idx_v, rows_v, sem):
        wid = lax.axis_index("s") * 2 + lax.axis_index("c")
        base = wid * b_per_w
        pltpu.sync_copy(idx_hbm.at[pl.ds(base, b_per_w)], idx_v)
        pltpu.async_copy(table_hbm.at[idx_v], rows_v, sem).wait()
        pltpu.sync_copy(rows_v, out_hbm.at[pl.ds(base, b_per_w)])

    return k(table, idx)


def kernel(x, mean, stddev, embeddings):
    b, h, w, c = x.shape
    emb_dim, num_emb = embeddings.shape
    xt = jnp.transpose(x, (0, 3, 1, 2)).reshape(-1, emb_dim)
    embt = embeddings.T
    ms = jnp.stack([mean[0], stddev[0]]).reshape(1, 2)
    idx3, table = _tc_argmin(ms, xt, embeddings, embt)
    idx = idx3.reshape(-1)
    q = _sc_gather(table, idx)[:, :emb_dim]
    quantised = q.reshape(b, c, h, w).transpose(0, 2, 3, 1)
    disc_out = idx.reshape(-1, c)
    return quantised, disc_out
